# baseline (speedup 1.0000x reference)
import jax
import jax.numpy as jnp
from jax import lax
from jax.experimental import pallas as pl
from jax.experimental.pallas import tpu as pltpu
from jax.experimental.pallas import tpu_sc as plsc

_N = 10000
_E = 320000
_D = 128
_R = 8

_NC = 2
_NS = 16
_C = 128
_CPT = 158
_EPAD = _NS * _CPT * _C
_DH = _D // _NC

_ACC_ROWS = 10240
_ORPS = _ACC_ROWS // _NS
_OC = 128


_BN = 1000


def _proj_body(x_ref, w_ref, loop_ref, b_ref, proj_ref, sl_ref):
    x = x_ref[...]
    for r in range(_R):
        proj_ref[r] = jnp.dot(x, w_ref[r], preferred_element_type=jnp.float32)
    sl = jnp.dot(x, loop_ref[...],
                 preferred_element_type=jnp.float32) + b_ref[...]
    sl_ref[0] = sl[:, :_DH]
    sl_ref[1] = sl[:, _DH:]


def _proj_call(x, w, loop_w, b2d):
    return pl.pallas_call(
        _proj_body,
        grid=(_N // _BN,),
        in_specs=[pl.BlockSpec((_BN, _D), lambda i: (i, 0)),
                  pl.BlockSpec((_R, _D, _D), lambda i: (0, 0, 0)),
                  pl.BlockSpec((_D, _D), lambda i: (0, 0)),
                  pl.BlockSpec((1, _D), lambda i: (0, 0))],
        out_specs=[pl.BlockSpec((_R, _BN, _D), lambda i: (0, i, 0)),
                   pl.BlockSpec((_NC, _BN, _DH), lambda i: (0, i, 0))],
        out_shape=[jax.ShapeDtypeStruct((_R, _N, _D), jnp.float32),
                   jax.ShapeDtypeStruct((_NC, _N, _DH), jnp.float32)],
    )(x, w, loop_w, b2d)


def _edge_body(flat_h, dst_h, sl_h, proj_h, out_h,
               gidx_v, dst_v, rows0_v, rows1_v, acc_sh, sem0, sem1):
    cid = lax.axis_index("c")
    sid = lax.axis_index("s")

    for k in range(_ORPS // _OC):
        base = sid * _ORPS + k * _OC

        @pl.when(base + _OC <= _N)
        def _():
            pltpu.sync_copy(sl_h.at[cid, pl.ds(base, _OC)],
                            rows0_v.at[pl.ds(0, _OC)])
            pltpu.sync_copy(rows0_v.at[pl.ds(0, _OC)],
                            acc_sh.at[pl.ds(base, _OC)])

        @pl.when(jnp.logical_and(base < _N, base + _OC > _N))
        def _():
            pltpu.sync_copy(sl_h.at[cid, pl.ds(base, _N % _OC)],
                            rows0_v.at[pl.ds(0, _N % _OC)])
            pltpu.sync_copy(rows0_v.at[pl.ds(0, _N % _OC)],
                            acc_sh.at[pl.ds(base, _N % _OC)])

    plsc.subcore_barrier()

    pltpu.sync_copy(flat_h.at[sid], gidx_v.at[pl.ds(0, _CPT)])
    pltpu.sync_copy(dst_h.at[sid], dst_v)

    def cidx(j):
        for i in range(_C // 16):
            sl = pl.ds(i * 16, 16)
            gidx_v[j, sl] = gidx_v[j, sl] * _NC + cid

    cidx(0)
    cidx(1)
    pltpu.async_copy(proj_h.at[gidx_v.at[0]], rows0_v, sem0)

    def chunk2(jj, carry):
        j0 = 2 * jj
        j1 = j0 + 1
        pltpu.async_copy(proj_h.at[gidx_v.at[j1]], rows1_v, sem1)
        cidx(j0 + 2)
        cidx(j1 + 2)
        pltpu.make_async_copy(proj_h.at[pl.ds(0, _C)], rows0_v, sem0).wait()
        pltpu.sync_copy(rows0_v, acc_sh.at[dst_v.at[j0]], add=True)

        @pl.when(jj < _CPT // 2 - 1)
        def _():
            pltpu.async_copy(proj_h.at[gidx_v.at[j0 + 2]], rows0_v, sem0)

        pltpu.make_async_copy(proj_h.at[pl.ds(0, _C)], rows1_v, sem1).wait()
        pltpu.sync_copy(rows1_v, acc_sh.at[dst_v.at[j1]], add=True)
        return carry

    lax.fori_loop(0, _CPT // 2, chunk2, 0)
    plsc.subcore_barrier()

    zero16 = jnp.zeros((16,), jnp.float32)

    def drain(nrows, base):
        pltpu.sync_copy(acc_sh.at[pl.ds(base, nrows)],
                        rows0_v.at[pl.ds(0, nrows)])

        def relu_row(t, carry):
            sl = pl.ds((t % 4) * 16, 16)
            rows0_v[t // 4, sl] = jnp.maximum(rows0_v[t // 4, sl], zero16)
            return carry

        lax.fori_loop(0, nrows * _DH // 16, relu_row, 0)
        pltpu.sync_copy(rows0_v.at[pl.ds(0, nrows)],
                        out_h.at[pl.ds(base, nrows), cid])

    for k in range(_ORPS // _OC):
        base = sid * _ORPS + k * _OC

        @pl.when(base + _OC <= _N)
        def _():
            drain(_OC, base)

        @pl.when(jnp.logical_and(base < _N, base + _OC > _N))
        def _():
            drain(_N % _OC, base)


_edge_call = pl.kernel(
    _edge_body,
    out_type=jax.ShapeDtypeStruct((_N, _NC, _DH), jnp.float32),
    mesh=plsc.VectorSubcoreMesh(core_axis_name="c", subcore_axis_name="s"),
    scratch_types=[
        pltpu.VMEM((_CPT + 2, _C), jnp.int32),
        pltpu.VMEM((_CPT, _C), jnp.int32),
        pltpu.VMEM((_C, _DH), jnp.float32),
        pltpu.VMEM((_C, _DH), jnp.float32),
        pltpu.VMEM_SHARED((_ACC_ROWS, _DH), jnp.float32),
        pltpu.SemaphoreType.DMA,
        pltpu.SemaphoreType.DMA,
    ],
    compiler_params=pltpu.CompilerParams(use_tc_tiling_on_sc=False),
)


def kernel(feat, edge_index, etype, W1, loop1, b1, W2, loop2, b2):
    src = edge_index[0]
    dst = edge_index[1]
    pad = _EPAD - _E
    flat = etype * _N + src
    flat3 = jnp.concatenate([flat, jnp.zeros((pad,), jnp.int32)]
                            ).reshape(_NS, _CPT, _C)
    dst3 = jnp.concatenate([dst, jnp.full((pad,), _N, jnp.int32)]
                           ).reshape(_NS, _CPT, _C)

    proj1, sl1 = _proj_call(feat, W1, loop1, b1.reshape(1, _D))
    h1 = _edge_call(flat3, dst3, sl1, proj1.reshape(_R * _N * _NC, _DH))
    proj2, sl2 = _proj_call(h1.reshape(_N, _D), W2, loop2, b2.reshape(1, _D))
    out = _edge_call(flat3, dst3, sl2, proj2.reshape(_R * _N * _NC, _DH))
    return out.reshape(_N, _D)


# speedup vs baseline: 1.1680x; 1.1680x over previous
import jax
import jax.numpy as jnp
from jax import lax
from jax.experimental import pallas as pl
from jax.experimental.pallas import tpu as pltpu
from jax.experimental.pallas import tpu_sc as plsc

_N = 10000
_E = 320000
_D = 128
_R = 8

_NC = 2
_NS = 16
_C = 128
_CPT = 158
_EPAD = _NS * _CPT * _C
_DH = _D // _NC

_ACC_ROWS = 10240
_ZB = 64
_ORPS = _ACC_ROWS // _NS
_OC = 128


_BN = 1000


def _proj_body(x_ref, w_ref, loop_ref, b_ref, proj_ref, sl_ref):
    x = x_ref[...]
    for r in range(_R):
        proj_ref[r] = jnp.dot(x, w_ref[r], preferred_element_type=jnp.float32)
    sl_ref[...] = jnp.dot(x, loop_ref[...],
                          preferred_element_type=jnp.float32) + b_ref[...]


def _combine_proj_body(p_ref, sl_ref, w_ref, loop_ref, b_ref,
                       proj_ref, sl2_ref):
    agg = jnp.concatenate([p_ref[0], p_ref[1]], axis=-1)
    h = jnp.maximum(agg + sl_ref[...], 0.0)
    for r in range(_R):
        proj_ref[r] = jnp.dot(h, w_ref[r], preferred_element_type=jnp.float32)
    sl2_ref[...] = jnp.dot(h, loop_ref[...],
                           preferred_element_type=jnp.float32) + b_ref[...]


def _final_body(p_ref, sl_ref, out_ref):
    agg = jnp.concatenate([p_ref[0], p_ref[1]], axis=-1)
    out_ref[...] = jnp.maximum(agg + sl_ref[...], 0.0)


def _x_spec():
    return pl.BlockSpec((_BN, _D), lambda i: (i, 0))


def _parts_spec():
    return pl.BlockSpec((_NC, _BN, _DH), lambda i: (0, i, 0))


def _w_spec():
    return pl.BlockSpec((_R, _D, _D), lambda i: (0, 0, 0))


def _loop_spec():
    return pl.BlockSpec((_D, _D), lambda i: (0, 0))


def _b_spec():
    return pl.BlockSpec((1, _D), lambda i: (0, 0))


def _proj_spec():
    return pl.BlockSpec((_R, _BN, _D), lambda i: (0, i, 0))


_GRID = (_N // _BN,)


def _proj_call(x, w, loop_w, b2d):
    return pl.pallas_call(
        _proj_body,
        grid=_GRID,
        in_specs=[_x_spec(), _w_spec(), _loop_spec(), _b_spec()],
        out_specs=[_proj_spec(), _x_spec()],
        out_shape=[jax.ShapeDtypeStruct((_R, _N, _D), jnp.float32),
                   jax.ShapeDtypeStruct((_N, _D), jnp.float32)],
    )(x, w, loop_w, b2d)


def _combine_proj_call(parts, sl, w, loop_w, b2d):
    return pl.pallas_call(
        _combine_proj_body,
        grid=_GRID,
        in_specs=[_parts_spec(), _x_spec(), _w_spec(), _loop_spec(), _b_spec()],
        out_specs=[_proj_spec(), _x_spec()],
        out_shape=[jax.ShapeDtypeStruct((_R, _N, _D), jnp.float32),
                   jax.ShapeDtypeStruct((_N, _D), jnp.float32)],
    )(parts, sl, w, loop_w, b2d)


def _final_call(parts, sl):
    return pl.pallas_call(
        _final_body,
        grid=_GRID,
        in_specs=[_parts_spec(), _x_spec()],
        out_specs=_x_spec(),
        out_shape=jax.ShapeDtypeStruct((_N, _D), jnp.float32),
    )(parts, sl)


def _edge_body(flat_h, dst_h, proj_h, parts_h,
               gidx_v, dst_v, rows0_v, rows1_v, acc_sh, sem0, sem1):
    cid = lax.axis_index("c")
    sid = lax.axis_index("s")

    zvec = jnp.zeros((16,), jnp.float32)

    def zrow(t, carry):
        rows0_v[t // 4, pl.ds((t % 4) * 16, 16)] = zvec
        return carry

    lax.fori_loop(0, _C * _DH // 16, zrow, 0)
    for k in range(_ORPS // _C):
        base = sid * _ORPS + k * _C
        pltpu.sync_copy(rows0_v, acc_sh.at[pl.ds(base, _C)])
    plsc.subcore_barrier()

    pltpu.sync_copy(flat_h.at[sid], gidx_v.at[pl.ds(0, _CPT)])
    pltpu.sync_copy(dst_h.at[sid], dst_v)

    def cidx(j):
        for i in range(_C // 16):
            sl = pl.ds(i * 16, 16)
            gidx_v[j, sl] = gidx_v[j, sl] * _NC + cid

    cidx(0)
    cidx(1)
    pltpu.async_copy(proj_h.at[gidx_v.at[0]], rows0_v, sem0)

    def chunk2(jj, carry):
        j0 = 2 * jj
        j1 = j0 + 1
        pltpu.async_copy(proj_h.at[gidx_v.at[j1]], rows1_v, sem1)
        cidx(j0 + 2)
        cidx(j1 + 2)
        pltpu.make_async_copy(proj_h.at[pl.ds(0, _C)], rows0_v, sem0).wait()
        pltpu.sync_copy(rows0_v, acc_sh.at[dst_v.at[j0]], add=True)

        @pl.when(jj < _CPT // 2 - 1)
        def _():
            pltpu.async_copy(proj_h.at[gidx_v.at[j0 + 2]], rows0_v, sem0)

        pltpu.make_async_copy(proj_h.at[pl.ds(0, _C)], rows1_v, sem1).wait()
        pltpu.sync_copy(rows1_v, acc_sh.at[dst_v.at[j1]], add=True)
        return carry

    lax.fori_loop(0, _CPT // 2, chunk2, 0)
    plsc.subcore_barrier()

    for k in range(_ORPS // _OC):
        base = sid * _ORPS + k * _OC
        pltpu.sync_copy(acc_sh.at[pl.ds(base, _OC)], rows0_v.at[pl.ds(0, _OC)])
        pltpu.sync_copy(rows0_v.at[pl.ds(0, _OC)],
                        parts_h.at[cid, pl.ds(base, _OC)])


_edge_call = pl.kernel(
    _edge_body,
    out_type=jax.ShapeDtypeStruct((_NC, _ACC_ROWS, _DH), jnp.float32),
    mesh=plsc.VectorSubcoreMesh(core_axis_name="c", subcore_axis_name="s"),
    scratch_types=[
        pltpu.VMEM((_CPT + 2, _C), jnp.int32),
        pltpu.VMEM((_CPT, _C), jnp.int32),
        pltpu.VMEM((_C, _DH), jnp.float32),
        pltpu.VMEM((_C, _DH), jnp.float32),
        pltpu.VMEM_SHARED((_ACC_ROWS, _DH), jnp.float32),
        pltpu.SemaphoreType.DMA,
        pltpu.SemaphoreType.DMA,
    ],
    compiler_params=pltpu.CompilerParams(use_tc_tiling_on_sc=False),
)


def kernel(feat, edge_index, etype, W1, loop1, b1, W2, loop2, b2):
    src = edge_index[0]
    dst = edge_index[1]
    pad = _EPAD - _E
    flat = etype * _N + src
    flat3 = jnp.concatenate([flat, jnp.zeros((pad,), jnp.int32)]
                            ).reshape(_NS, _CPT, _C)
    dst3 = jnp.concatenate([dst, jnp.full((pad,), _N, jnp.int32)]
                           ).reshape(_NS, _CPT, _C)

    proj1, sl1 = _proj_call(feat, W1, loop1, b1.reshape(1, _D))
    parts1 = _edge_call(flat3, dst3, proj1.reshape(_R * _N * _NC, _DH))
    proj2, sl2 = _combine_proj_call(parts1, sl1, W2, loop2, b2.reshape(1, _D))
    parts2 = _edge_call(flat3, dst3, proj2.reshape(_R * _N * _NC, _DH))
    return _final_call(parts2, sl2)
